# baseline (speedup 1.0000x reference)
import functools

import jax
import jax.numpy as jnp
from jax.experimental import pallas as pl

N = 10000
E = 160000
IN = 256
HEADS = 8
HID = 64
TH = 512
OUT = 40
PROJ = 256
HOPS = 2
TAU = 0.8

_BN = 1000


def _nspec(k, cols):
    return pl.BlockSpec((_BN, cols), lambda i: (i, 0))


def _wspec(shape):
    return pl.BlockSpec(shape, lambda i: (0,) * len(shape))


def _k_stage1(x_ref, wres_ref, bres_ref, wg1_ref, xp_ref, xproj_ref):
    xb = x_ref[...]
    xp_ref[...] = jnp.dot(xb, wg1_ref[...], preferred_element_type=jnp.float32)
    xproj_ref[...] = (
        jnp.dot(xb, wres_ref[...], preferred_element_type=jnp.float32)
        + bres_ref[...])


def _stage1(x, W_res, b_res, W_g1):
    return pl.pallas_call(
        _k_stage1,
        grid=(N // _BN,),
        in_specs=[_nspec(IN, IN), _wspec((IN, TH)), _wspec((1, TH)),
                  _wspec((IN, TH))],
        out_specs=[_nspec(TH, TH), _nspec(TH, TH)],
        out_shape=[jax.ShapeDtypeStruct((N, TH), jnp.float32),
                   jax.ShapeDtypeStruct((N, TH), jnp.float32)],
    )(x, W_res, b_res.reshape(1, TH), W_g1)


def _nspec2(cols):
    return pl.BlockSpec((_BN, cols), lambda i: (i, 0))


def _k_mm(a_ref, w_ref, o_ref):
    o_ref[...] = jnp.dot(a_ref[...], w_ref[...],
                         preferred_element_type=jnp.float32)


def _stage2(h1, W_g2):
    return pl.pallas_call(
        _k_mm,
        grid=(N // _BN,),
        in_specs=[_nspec2(TH), _wspec((TH, TH))],
        out_specs=_nspec2(TH),
        out_shape=jax.ShapeDtypeStruct((N, TH), jnp.float32),
    )(h1, W_g2)


def _k_stage3(h_ref, wp_ref, bp_ref, wbil_ref, we_ref, be_ref,
              hp_ref, q_ref, xl_ref):
    h = h_ref[...]
    hp = jnp.dot(h, wp_ref[...], preferred_element_type=jnp.float32) + bp_ref[...]
    hp_ref[...] = hp
    q_ref[...] = jnp.dot(hp, wbil_ref[...], preferred_element_type=jnp.float32)
    xl_ref[...] = (
        jnp.dot(h, we_ref[...], preferred_element_type=jnp.float32) + be_ref[...])


def _stage3(h_base, W_p, b_p, W_bil, W_e, b_e):
    return pl.pallas_call(
        _k_stage3,
        grid=(N // _BN,),
        in_specs=[_nspec2(TH), _wspec((TH, PROJ)), _wspec((1, PROJ)),
                  _wspec((PROJ, PROJ)), _wspec((TH, TH)), _wspec((1, TH))],
        out_specs=[_nspec2(PROJ), _nspec2(PROJ), _nspec2(TH)],
        out_shape=[jax.ShapeDtypeStruct((N, PROJ), jnp.float32),
                   jax.ShapeDtypeStruct((N, PROJ), jnp.float32),
                   jax.ShapeDtypeStruct((N, TH), jnp.float32)],
    )(h_base, W_p, b_p.reshape(1, PROJ), W_bil, W_e, b_e.reshape(1, TH))


_EG = 125
_EB = 10
_EC = 128


def _k_scores(qr_ref, hc_ref, l0_ref, l1_ref):
    s = jax.nn.sigmoid(jnp.sum(qr_ref[...] * hc_ref[...], axis=-1))
    l0_ref[...] = jnp.log(jnp.maximum(1.0 - s, 1e-9))
    l1_ref[...] = jnp.log(jnp.maximum(s, 1e-9))


def _scores(q_row, hp_col):
    return pl.pallas_call(
        _k_scores,
        grid=(_EG,),
        in_specs=[pl.BlockSpec((1, _EB, _EC, PROJ), lambda i: (i, 0, 0, 0))] * 2,
        out_specs=[pl.BlockSpec((1, _EB, _EC), lambda i: (i, 0, 0))] * 2,
        out_shape=[jax.ShapeDtypeStruct((_EG, _EB, _EC), jnp.float32)] * 2,
    )(q_row.reshape(_EG, _EB, _EC, PROJ), hp_col.reshape(_EG, _EB, _EC, PROJ))


def _k_stage4(h_ref, msg_ref, we_ref, be_ref, h2_ref, xl_ref):
    m = msg_ref[...]
    h2 = h_ref[...] + jnp.where(m > 0, m, jnp.exp(m) - 1.0)
    h2_ref[...] = h2
    xl_ref[...] = (
        jnp.dot(h2, we_ref[...], preferred_element_type=jnp.float32) + be_ref[...])


def _stage4(h, msg, W_e, b_e):
    return pl.pallas_call(
        _k_stage4,
        grid=(N // _BN,),
        in_specs=[_nspec2(TH), _nspec2(TH), _wspec((TH, TH)), _wspec((1, TH))],
        out_specs=[_nspec2(TH), _nspec2(TH)],
        out_shape=[jax.ShapeDtypeStruct((N, TH), jnp.float32),
                   jax.ShapeDtypeStruct((N, TH), jnp.float32)],
    )(h, msg, W_e, b_e.reshape(1, TH))


def _k_stage5(h_ref, msg_ref, wc1_ref, bc1_ref, bng_ref, bnb_ref, wc2_ref,
              bc2_ref, wl1a_ref, wl1b_ref, he_ref, logp_ref, ur_ref, uc_ref):
    m = msg_ref[...]
    h = h_ref[...] + jnp.where(m > 0, m, jnp.exp(m) - 1.0)
    he_ref[...] = h
    c = jnp.dot(h, wc1_ref[...], preferred_element_type=jnp.float32) + bc1_ref[...]
    c = c / jnp.sqrt(1.0 + 1e-5) * bng_ref[...] + bnb_ref[...]
    c = jnp.maximum(c, 0.0)
    lg = jnp.dot(c, wc2_ref[...], preferred_element_type=jnp.float32) + bc2_ref[...]
    lmax = jnp.max(lg, axis=-1, keepdims=True)
    sh = lg - lmax
    logp_ref[...] = sh - jnp.log(jnp.sum(jnp.exp(sh), axis=-1, keepdims=True))
    ur_ref[...] = jnp.dot(h, wl1a_ref[...], preferred_element_type=jnp.float32)
    uc_ref[...] = jnp.dot(h, wl1b_ref[...], preferred_element_type=jnp.float32)


def _stage5(h, msg, W_c1, b_c1, bn_g, bn_b, W_c2, b_c2, W_l1a, W_l1b):
    return pl.pallas_call(
        _k_stage5,
        grid=(N // _BN,),
        in_specs=[_nspec2(TH), _nspec2(TH), _wspec((TH, 4 * HID)),
                  _wspec((1, 4 * HID)), _wspec((1, 4 * HID)), _wspec((1, 4 * HID)),
                  _wspec((4 * HID, OUT)), _wspec((1, OUT)),
                  _wspec((TH, HID)), _wspec((TH, HID))],
        out_specs=[_nspec2(TH), _nspec2(OUT), _nspec2(HID), _nspec2(HID)],
        out_shape=[jax.ShapeDtypeStruct((N, TH), jnp.float32),
                   jax.ShapeDtypeStruct((N, OUT), jnp.float32),
                   jax.ShapeDtypeStruct((N, HID), jnp.float32),
                   jax.ShapeDtypeStruct((N, HID), jnp.float32)],
    )(h, msg, W_c1, b_c1.reshape(1, 4 * HID), bn_g.reshape(1, 4 * HID),
      bn_b.reshape(1, 4 * HID), W_c2, b_c2.reshape(1, OUT), W_l1a, W_l1b)


def _k_link(ur_ref, uc_ref, bl1_ref, wl2_ref, bl2_ref, out_ref):
    v = jnp.maximum(ur_ref[...] + uc_ref[...] + bl1_ref[...], 0.0)
    out_ref[...] = jax.nn.sigmoid(
        jnp.sum(v * wl2_ref[...], axis=-1) + bl2_ref[0, 0, 0, 0])


def _link(ur_row, uc_col, b_l1, W_l2, b_l2):
    return pl.pallas_call(
        _k_link,
        grid=(_EG,),
        in_specs=[pl.BlockSpec((1, _EB, _EC, HID), lambda i: (i, 0, 0, 0))] * 2
        + [_wspec((1, 1, 1, HID)), _wspec((1, 1, 1, HID)), _wspec((1, 1, 1, 1))],
        out_specs=pl.BlockSpec((1, _EB, _EC), lambda i: (i, 0, 0)),
        out_shape=jax.ShapeDtypeStruct((_EG, _EB, _EC), jnp.float32),
    )(ur_row.reshape(_EG, _EB, _EC, HID), uc_col.reshape(_EG, _EB, _EC, HID),
      b_l1.reshape(1, 1, 1, HID), W_l2.reshape(1, 1, 1, HID),
      b_l2.reshape(1, 1, 1, 1))


def _ln(x, g, b):
    mu = jnp.mean(x, axis=-1, keepdims=True)
    var = jnp.var(x, axis=-1, keepdims=True)
    return (x - mu) / jnp.sqrt(var + 1e-5) * g + b


def _gat_edge(xp, a_s, a_d, row2, col2):
    xph = xp.reshape(N, HEADS, HID)
    asrc = jnp.sum(xph * a_s, axis=-1)
    adst = jnp.sum(xph * a_d, axis=-1)
    alpha = jax.nn.leaky_relu(asrc[row2] + adst[col2], 0.2)
    amax = jax.ops.segment_max(alpha, col2, num_segments=N)
    amax = jnp.where(jnp.isfinite(amax), amax, 0.0)
    e = jnp.exp(alpha - amax[col2])
    den = jax.ops.segment_sum(e, col2, num_segments=N)
    a = e / (den[col2] + 1e-16)
    msg = a[:, :, None] * xph[row2]
    agg = jax.ops.segment_sum(msg, col2, num_segments=N)
    return agg.reshape(N, TH)


def kernel(x, W_res, b_res, emb, W_g1, a_s1, a_d1, b_g1, ln1_g, ln1_b,
           W_g2, a_s2, a_d2, b_g2, ln2_g, ln2_b, W_p, b_p, W_bil, W_e, b_e,
           W_l1, b_l1, W_l2, b_l2, W_c1, b_c1, bn_g, bn_b, W_c2, b_c2,
           edge_index, y_label, target_mask):
    row = edge_index[0].astype(jnp.int32)
    col = edge_index[1].astype(jnp.int32)
    loop = jnp.arange(N, dtype=jnp.int32)
    row2 = jnp.concatenate([row, loop])
    col2 = jnp.concatenate([col, loop])

    xp1, x_proj = _stage1(x, W_res, b_res, W_g1)
    agg1 = _gat_edge(xp1, a_s1, a_d1, row2, col2)
    h1 = jax.nn.elu(_ln(agg1 + b_g1, ln1_g, ln1_b))

    xp2 = _stage2(h1, W_g2)
    agg2 = _gat_edge(xp2, a_s2, a_d2, row2, col2)
    h_base = jax.nn.elu(_ln(agg2 + b_g2, ln2_g, ln2_b) + x_proj)

    hp, q, xl1 = _stage3(h_base, W_p, b_p, W_bil, W_e, b_e)

    l0, l1 = _scores(q[row], hp[col])
    l0 = l0.reshape(E)
    l1 = l1.reshape(E)

    gkey = jax.random.key(42)
    final_w = jnp.zeros((E,), jnp.float32)
    active = target_mask.astype(jnp.float32)
    for hstep in range(HOPS):
        g = jax.random.gumbel(jax.random.fold_in(gkey, hstep), (E, 2))
        samp = ((l1 + g[:, 1]) > (l0 + g[:, 0])).astype(jnp.float32)
        cur = samp * active[row]
        final_w = jnp.maximum(final_w, cur)
        new_active = jax.ops.segment_sum(cur, col, num_segments=N)
        active = (new_active > 1e-5).astype(jnp.float32)
    weights = final_w

    wmax = jax.ops.segment_max(weights, col, num_segments=N)
    wmax = jnp.where(jnp.isfinite(wmax), wmax, 0.0)
    ew = jnp.exp(weights - wmax[col])
    dw = jax.ops.segment_sum(ew, col, num_segments=N)
    nw = ew / (dw[col] + 1e-16)

    msg1 = jax.ops.segment_sum(nw[:, None] * xl1[row], col, num_segments=N)
    h2, xl2 = _stage4(h_base, msg1, W_e, b_e)
    msg2 = jax.ops.segment_sum(nw[:, None] * xl2[row], col, num_segments=N)

    h_enh, logp, ur, uc = _stage5(
        h2, msg2, W_c1, b_c1, bn_g, bn_b, W_c2, b_c2,
        W_l1[:TH], W_l1[TH:])

    link = _link(ur[row], uc[col], b_l1, W_l2, b_l2).reshape(E)

    return (logp, link, h_enh, weights)


# speedup vs baseline: 1.0083x; 1.0083x over previous
import functools

import jax
import jax.numpy as jnp
from jax import lax
from jax.experimental import pallas as pl
from jax.experimental.pallas import tpu as pltpu
from jax.experimental.pallas import tpu_sc as plsc

N = 10000
E = 160000
IN = 256
HEADS = 8
HID = 64
TH = 512
OUT = 40
PROJ = 256
HOPS = 2
TAU = 0.8

_BN = 1000


def _nspec(k, cols):
    return pl.BlockSpec((_BN, cols), lambda i: (i, 0))


def _wspec(shape):
    return pl.BlockSpec(shape, lambda i: (0,) * len(shape))


def _k_stage1(x_ref, wres_ref, bres_ref, wg1_ref, xp_ref, xproj_ref):
    xb = x_ref[...]
    xp_ref[...] = jnp.dot(xb, wg1_ref[...], preferred_element_type=jnp.float32)
    xproj_ref[...] = (
        jnp.dot(xb, wres_ref[...], preferred_element_type=jnp.float32)
        + bres_ref[...])


def _stage1(x, W_res, b_res, W_g1):
    return pl.pallas_call(
        _k_stage1,
        grid=(N // _BN,),
        in_specs=[_nspec(IN, IN), _wspec((IN, TH)), _wspec((1, TH)),
                  _wspec((IN, TH))],
        out_specs=[_nspec(TH, TH), _nspec(TH, TH)],
        out_shape=[jax.ShapeDtypeStruct((N, TH), jnp.float32),
                   jax.ShapeDtypeStruct((N, TH), jnp.float32)],
    )(x, W_res, b_res.reshape(1, TH), W_g1)


def _nspec2(cols):
    return pl.BlockSpec((_BN, cols), lambda i: (i, 0))


def _k_mm(a_ref, w_ref, o_ref):
    o_ref[...] = jnp.dot(a_ref[...], w_ref[...],
                         preferred_element_type=jnp.float32)


def _stage2(h1, W_g2):
    return pl.pallas_call(
        _k_mm,
        grid=(N // _BN,),
        in_specs=[_nspec2(TH), _wspec((TH, TH))],
        out_specs=_nspec2(TH),
        out_shape=jax.ShapeDtypeStruct((N, TH), jnp.float32),
    )(h1, W_g2)


def _k_stage3(h_ref, wp_ref, bp_ref, wbil_ref, we_ref, be_ref,
              hp_ref, q_ref, xl_ref):
    h = h_ref[...]
    hp = jnp.dot(h, wp_ref[...], preferred_element_type=jnp.float32) + bp_ref[...]
    hp_ref[...] = hp
    q_ref[...] = jnp.dot(hp, wbil_ref[...], preferred_element_type=jnp.float32)
    xl_ref[...] = (
        jnp.dot(h, we_ref[...], preferred_element_type=jnp.float32) + be_ref[...])


def _stage3(h_base, W_p, b_p, W_bil, W_e, b_e):
    return pl.pallas_call(
        _k_stage3,
        grid=(N // _BN,),
        in_specs=[_nspec2(TH), _wspec((TH, PROJ)), _wspec((1, PROJ)),
                  _wspec((PROJ, PROJ)), _wspec((TH, TH)), _wspec((1, TH))],
        out_specs=[_nspec2(PROJ), _nspec2(PROJ), _nspec2(TH)],
        out_shape=[jax.ShapeDtypeStruct((N, PROJ), jnp.float32),
                   jax.ShapeDtypeStruct((N, PROJ), jnp.float32),
                   jax.ShapeDtypeStruct((N, TH), jnp.float32)],
    )(h_base, W_p, b_p.reshape(1, PROJ), W_bil, W_e, b_e.reshape(1, TH))


_EG = 125
_EB = 10
_EC = 128


def _k_scores(qr_ref, hc_ref, l0_ref, l1_ref):
    s = jax.nn.sigmoid(jnp.sum(qr_ref[...] * hc_ref[...], axis=-1))
    l0_ref[...] = jnp.log(jnp.maximum(1.0 - s, 1e-9))
    l1_ref[...] = jnp.log(jnp.maximum(s, 1e-9))


def _scores(q_row, hp_col):
    return pl.pallas_call(
        _k_scores,
        grid=(_EG,),
        in_specs=[pl.BlockSpec((1, _EB, _EC, PROJ), lambda i: (i, 0, 0, 0))] * 2,
        out_specs=[pl.BlockSpec((1, _EB, _EC), lambda i: (i, 0, 0))] * 2,
        out_shape=[jax.ShapeDtypeStruct((_EG, _EB, _EC), jnp.float32)] * 2,
    )(q_row.reshape(_EG, _EB, _EC, PROJ), hp_col.reshape(_EG, _EB, _EC, PROJ))


def _k_stage4(h_ref, msg_ref, we_ref, be_ref, h2_ref, xl_ref):
    m = msg_ref[...]
    h2 = h_ref[...] + jnp.where(m > 0, m, jnp.exp(m) - 1.0)
    h2_ref[...] = h2
    xl_ref[...] = (
        jnp.dot(h2, we_ref[...], preferred_element_type=jnp.float32) + be_ref[...])


def _stage4(h, msg, W_e, b_e):
    return pl.pallas_call(
        _k_stage4,
        grid=(N // _BN,),
        in_specs=[_nspec2(TH), _nspec2(TH), _wspec((TH, TH)), _wspec((1, TH))],
        out_specs=[_nspec2(TH), _nspec2(TH)],
        out_shape=[jax.ShapeDtypeStruct((N, TH), jnp.float32),
                   jax.ShapeDtypeStruct((N, TH), jnp.float32)],
    )(h, msg, W_e, b_e.reshape(1, TH))


def _k_stage5(h_ref, msg_ref, wc1_ref, bc1_ref, bng_ref, bnb_ref, wc2_ref,
              bc2_ref, wl1a_ref, wl1b_ref, he_ref, logp_ref, ur_ref, uc_ref):
    m = msg_ref[...]
    h = h_ref[...] + jnp.where(m > 0, m, jnp.exp(m) - 1.0)
    he_ref[...] = h
    c = jnp.dot(h, wc1_ref[...], preferred_element_type=jnp.float32) + bc1_ref[...]
    c = c / jnp.sqrt(1.0 + 1e-5) * bng_ref[...] + bnb_ref[...]
    c = jnp.maximum(c, 0.0)
    lg = jnp.dot(c, wc2_ref[...], preferred_element_type=jnp.float32) + bc2_ref[...]
    lmax = jnp.max(lg, axis=-1, keepdims=True)
    sh = lg - lmax
    logp_ref[...] = sh - jnp.log(jnp.sum(jnp.exp(sh), axis=-1, keepdims=True))
    ur_ref[...] = jnp.dot(h, wl1a_ref[...], preferred_element_type=jnp.float32)
    uc_ref[...] = jnp.dot(h, wl1b_ref[...], preferred_element_type=jnp.float32)


def _stage5(h, msg, W_c1, b_c1, bn_g, bn_b, W_c2, b_c2, W_l1a, W_l1b):
    return pl.pallas_call(
        _k_stage5,
        grid=(N // _BN,),
        in_specs=[_nspec2(TH), _nspec2(TH), _wspec((TH, 4 * HID)),
                  _wspec((1, 4 * HID)), _wspec((1, 4 * HID)), _wspec((1, 4 * HID)),
                  _wspec((4 * HID, OUT)), _wspec((1, OUT)),
                  _wspec((TH, HID)), _wspec((TH, HID))],
        out_specs=[_nspec2(TH), _nspec2(OUT), _nspec2(HID), _nspec2(HID)],
        out_shape=[jax.ShapeDtypeStruct((N, TH), jnp.float32),
                   jax.ShapeDtypeStruct((N, OUT), jnp.float32),
                   jax.ShapeDtypeStruct((N, HID), jnp.float32),
                   jax.ShapeDtypeStruct((N, HID), jnp.float32)],
    )(h, msg, W_c1, b_c1.reshape(1, 4 * HID), bn_g.reshape(1, 4 * HID),
      bn_b.reshape(1, 4 * HID), W_c2, b_c2.reshape(1, OUT), W_l1a, W_l1b)


def _k_link(ur_ref, uc_ref, bl1_ref, wl2_ref, bl2_ref, out_ref):
    v = jnp.maximum(ur_ref[...] + uc_ref[...] + bl1_ref[...], 0.0)
    out_ref[...] = jax.nn.sigmoid(
        jnp.sum(v * wl2_ref[...], axis=-1) + bl2_ref[0, 0, 0, 0])


def _link(ur_row, uc_col, b_l1, W_l2, b_l2):
    return pl.pallas_call(
        _k_link,
        grid=(_EG,),
        in_specs=[pl.BlockSpec((1, _EB, _EC, HID), lambda i: (i, 0, 0, 0))] * 2
        + [_wspec((1, 1, 1, HID)), _wspec((1, 1, 1, HID)), _wspec((1, 1, 1, 1))],
        out_specs=pl.BlockSpec((1, _EB, _EC), lambda i: (i, 0, 0)),
        out_shape=jax.ShapeDtypeStruct((_EG, _EB, _EC), jnp.float32),
    )(ur_row.reshape(_EG, _EB, _EC, HID), uc_col.reshape(_EG, _EB, _EC, HID),
      b_l1.reshape(1, 1, 1, HID), W_l2.reshape(1, 1, 1, HID),
      b_l2.reshape(1, 1, 1, 1))


_SC_CH = 128
_NSUB = 16
_FC = 128


_NP = 10240


def _sc_segsum_call(nchunks):
    mesh = plsc.VectorSubcoreMesh(core_axis_name="c", subcore_axis_name="s")
    rows_per_sub = _NP // _NSUB

    @functools.partial(
        pl.kernel, mesh=mesh,
        out_type=jax.ShapeDtypeStruct((4, _NP, _FC), jnp.float32),
        scratch_types=[
            pltpu.VMEM((nchunks, _SC_CH), jnp.int32),
            pltpu.VMEM((_SC_CH, _FC), jnp.float32),
            pltpu.VMEM_SHARED((_NP, _FC), jnp.float32),
        ],
    )
    def k(m_hbm, idx_hbm, zero_hbm, out_hbm, idx_v, rows_v, acc):
        c = lax.axis_index("c")
        s = lax.axis_index("s")
        pltpu.sync_copy(idx_hbm.at[s], idx_v)
        for j in range(2):
            fc = c * 2 + j
            pltpu.sync_copy(
                zero_hbm.at[pl.ds(s * rows_per_sub, rows_per_sub)],
                acc.at[pl.ds(s * rows_per_sub, rows_per_sub)])
            plsc.subcore_barrier()

            def body(t, carry):
                base = (s * nchunks + t) * _SC_CH
                pltpu.sync_copy(m_hbm.at[fc, pl.ds(base, _SC_CH)], rows_v)
                pltpu.sync_copy(rows_v, acc.at[idx_v.at[t]], add=True)
                return carry

            lax.fori_loop(0, nchunks, body, 0)
            plsc.subcore_barrier()
            pltpu.sync_copy(
                acc.at[pl.ds(s * rows_per_sub, rows_per_sub)],
                out_hbm.at[fc, pl.ds(s * rows_per_sub, rows_per_sub)])
            plsc.subcore_barrier()

    return k


def _sc_segsum(m, col):
    e = m.shape[0]
    ep = -(-e // (_NSUB * _SC_CH)) * (_NSUB * _SC_CH)
    nchunks = ep // (_NSUB * _SC_CH)
    pad = ep - e
    m4 = jnp.pad(m, ((0, pad), (0, 0))).reshape(ep, 4, _FC).transpose(1, 0, 2)
    idx3 = jnp.pad(col, (0, pad)).reshape(_NSUB, nchunks, _SC_CH)
    zero = jnp.zeros((_NP, _FC), jnp.float32)
    out4 = _sc_segsum_call(nchunks)(m4, idx3, zero)
    return out4[:, :N].transpose(1, 0, 2).reshape(N, TH)


def _ln(x, g, b):
    mu = jnp.mean(x, axis=-1, keepdims=True)
    var = jnp.var(x, axis=-1, keepdims=True)
    return (x - mu) / jnp.sqrt(var + 1e-5) * g + b


def _gat_edge(xp, a_s, a_d, row2, col2):
    xph = xp.reshape(N, HEADS, HID)
    asrc = jnp.sum(xph * a_s, axis=-1)
    adst = jnp.sum(xph * a_d, axis=-1)
    alpha = jax.nn.leaky_relu(asrc[row2] + adst[col2], 0.2)
    amax = jax.ops.segment_max(alpha, col2, num_segments=N)
    amax = jnp.where(jnp.isfinite(amax), amax, 0.0)
    e = jnp.exp(alpha - amax[col2])
    den = jax.ops.segment_sum(e, col2, num_segments=N)
    a = e / (den[col2] + 1e-16)
    msg = a[:, :, None] * xph[row2]
    agg = jax.ops.segment_sum(msg, col2, num_segments=N)
    return agg.reshape(N, TH)


def kernel(x, W_res, b_res, emb, W_g1, a_s1, a_d1, b_g1, ln1_g, ln1_b,
           W_g2, a_s2, a_d2, b_g2, ln2_g, ln2_b, W_p, b_p, W_bil, W_e, b_e,
           W_l1, b_l1, W_l2, b_l2, W_c1, b_c1, bn_g, bn_b, W_c2, b_c2,
           edge_index, y_label, target_mask):
    row = edge_index[0].astype(jnp.int32)
    col = edge_index[1].astype(jnp.int32)
    loop = jnp.arange(N, dtype=jnp.int32)
    row2 = jnp.concatenate([row, loop])
    col2 = jnp.concatenate([col, loop])

    xp1, x_proj = _stage1(x, W_res, b_res, W_g1)
    agg1 = _gat_edge(xp1, a_s1, a_d1, row2, col2)
    h1 = jax.nn.elu(_ln(agg1 + b_g1, ln1_g, ln1_b))

    xp2 = _stage2(h1, W_g2)
    agg2 = _gat_edge(xp2, a_s2, a_d2, row2, col2)
    h_base = jax.nn.elu(_ln(agg2 + b_g2, ln2_g, ln2_b) + x_proj)

    hp, q, xl1 = _stage3(h_base, W_p, b_p, W_bil, W_e, b_e)

    l0, l1 = _scores(q[row], hp[col])
    l0 = l0.reshape(E)
    l1 = l1.reshape(E)

    gkey = jax.random.key(42)
    final_w = jnp.zeros((E,), jnp.float32)
    active = target_mask.astype(jnp.float32)
    for hstep in range(HOPS):
        g = jax.random.gumbel(jax.random.fold_in(gkey, hstep), (E, 2))
        samp = ((l1 + g[:, 1]) > (l0 + g[:, 0])).astype(jnp.float32)
        cur = samp * active[row]
        final_w = jnp.maximum(final_w, cur)
        new_active = jax.ops.segment_sum(cur, col, num_segments=N)
        active = (new_active > 1e-5).astype(jnp.float32)
    weights = final_w

    wmax = jax.ops.segment_max(weights, col, num_segments=N)
    wmax = jnp.where(jnp.isfinite(wmax), wmax, 0.0)
    ew = jnp.exp(weights - wmax[col])
    dw = jax.ops.segment_sum(ew, col, num_segments=N)
    nw = ew / (dw[col] + 1e-16)

    msg1 = _sc_segsum(nw[:, None] * xl1[row], col)
    h2, xl2 = _stage4(h_base, msg1, W_e, b_e)
    msg2 = _sc_segsum(nw[:, None] * xl2[row], col)

    h_enh, logp, ur, uc = _stage5(
        h2, msg2, W_c1, b_c1, bn_g, bn_b, W_c2, b_c2,
        W_l1[:TH], W_l1[TH:])

    link = _link(ur[row], uc[col], b_l1, W_l2, b_l2).reshape(E)

    return (logp, link, h_enh, weights)


# speedup vs baseline: 2.5481x; 2.5270x over previous
import functools

import jax
import jax.numpy as jnp
from jax import lax
from jax.experimental import pallas as pl
from jax.experimental.pallas import tpu as pltpu
from jax.experimental.pallas import tpu_sc as plsc

N = 10000
E = 160000
IN = 256
HEADS = 8
HID = 64
TH = 512
OUT = 40
PROJ = 256
HOPS = 2
TAU = 0.8

_BN = 1000


def _nspec(k, cols):
    return pl.BlockSpec((_BN, cols), lambda i: (i, 0))


def _wspec(shape):
    return pl.BlockSpec(shape, lambda i: (0,) * len(shape))


def _k_stage1(x_ref, wres_ref, bres_ref, wg1_ref, xp_ref, xproj_ref):
    xb = x_ref[...]
    xp_ref[...] = jnp.dot(xb, wg1_ref[...], preferred_element_type=jnp.float32)
    xproj_ref[...] = (
        jnp.dot(xb, wres_ref[...], preferred_element_type=jnp.float32)
        + bres_ref[...])


def _stage1(x, W_res, b_res, W_g1):
    return pl.pallas_call(
        _k_stage1,
        grid=(N // _BN,),
        in_specs=[_nspec(IN, IN), _wspec((IN, TH)), _wspec((1, TH)),
                  _wspec((IN, TH))],
        out_specs=[_nspec(TH, TH), _nspec(TH, TH)],
        out_shape=[jax.ShapeDtypeStruct((N, TH), jnp.float32),
                   jax.ShapeDtypeStruct((N, TH), jnp.float32)],
    )(x, W_res, b_res.reshape(1, TH), W_g1)


def _nspec2(cols):
    return pl.BlockSpec((_BN, cols), lambda i: (i, 0))


def _k_mm(a_ref, w_ref, o_ref):
    o_ref[...] = jnp.dot(a_ref[...], w_ref[...],
                         preferred_element_type=jnp.float32)


def _stage2(h1, W_g2):
    return pl.pallas_call(
        _k_mm,
        grid=(N // _BN,),
        in_specs=[_nspec2(TH), _wspec((TH, TH))],
        out_specs=_nspec2(TH),
        out_shape=jax.ShapeDtypeStruct((N, TH), jnp.float32),
    )(h1, W_g2)


def _k_stage3(h_ref, wp_ref, bp_ref, wbil_ref, we_ref, be_ref,
              hp_ref, q_ref, xl_ref):
    h = h_ref[...]
    hp = jnp.dot(h, wp_ref[...], preferred_element_type=jnp.float32) + bp_ref[...]
    hp_ref[...] = hp
    q_ref[...] = jnp.dot(hp, wbil_ref[...], preferred_element_type=jnp.float32)
    xl_ref[...] = (
        jnp.dot(h, we_ref[...], preferred_element_type=jnp.float32) + be_ref[...])


def _stage3(h_base, W_p, b_p, W_bil, W_e, b_e):
    return pl.pallas_call(
        _k_stage3,
        grid=(N // _BN,),
        in_specs=[_nspec2(TH), _wspec((TH, PROJ)), _wspec((1, PROJ)),
                  _wspec((PROJ, PROJ)), _wspec((TH, TH)), _wspec((1, TH))],
        out_specs=[_nspec2(PROJ), _nspec2(PROJ), _nspec2(TH)],
        out_shape=[jax.ShapeDtypeStruct((N, PROJ), jnp.float32),
                   jax.ShapeDtypeStruct((N, PROJ), jnp.float32),
                   jax.ShapeDtypeStruct((N, TH), jnp.float32)],
    )(h_base, W_p, b_p.reshape(1, PROJ), W_bil, W_e, b_e.reshape(1, TH))


_EG = 125
_EB = 10
_EC = 128


def _k_scores(qr_ref, hc_ref, l0_ref, l1_ref):
    s = jax.nn.sigmoid(jnp.sum(qr_ref[...] * hc_ref[...], axis=-1))
    l0_ref[...] = jnp.log(jnp.maximum(1.0 - s, 1e-9))
    l1_ref[...] = jnp.log(jnp.maximum(s, 1e-9))


def _scores(q_row, hp_col):
    return pl.pallas_call(
        _k_scores,
        grid=(_EG,),
        in_specs=[pl.BlockSpec((1, _EB, _EC, PROJ), lambda i: (i, 0, 0, 0))] * 2,
        out_specs=[pl.BlockSpec((1, _EB, _EC), lambda i: (i, 0, 0))] * 2,
        out_shape=[jax.ShapeDtypeStruct((_EG, _EB, _EC), jnp.float32)] * 2,
    )(q_row.reshape(_EG, _EB, _EC, PROJ), hp_col.reshape(_EG, _EB, _EC, PROJ))


def _k_stage4(h_ref, msg_ref, we_ref, be_ref, h2_ref, xl_ref):
    m = msg_ref[...]
    h2 = h_ref[...] + jnp.where(m > 0, m, jnp.exp(m) - 1.0)
    h2_ref[...] = h2
    xl_ref[...] = (
        jnp.dot(h2, we_ref[...], preferred_element_type=jnp.float32) + be_ref[...])


def _stage4(h, msg, W_e, b_e):
    return pl.pallas_call(
        _k_stage4,
        grid=(N // _BN,),
        in_specs=[_nspec2(TH), _nspec2(TH), _wspec((TH, TH)), _wspec((1, TH))],
        out_specs=[_nspec2(TH), _nspec2(TH)],
        out_shape=[jax.ShapeDtypeStruct((N, TH), jnp.float32),
                   jax.ShapeDtypeStruct((N, TH), jnp.float32)],
    )(h, msg, W_e, b_e.reshape(1, TH))


def _k_stage5(h_ref, msg_ref, wc1_ref, bc1_ref, bng_ref, bnb_ref, wc2_ref,
              bc2_ref, wl1a_ref, wl1b_ref, he_ref, logp_ref, ur_ref, uc_ref):
    m = msg_ref[...]
    h = h_ref[...] + jnp.where(m > 0, m, jnp.exp(m) - 1.0)
    he_ref[...] = h
    c = jnp.dot(h, wc1_ref[...], preferred_element_type=jnp.float32) + bc1_ref[...]
    c = c / jnp.sqrt(1.0 + 1e-5) * bng_ref[...] + bnb_ref[...]
    c = jnp.maximum(c, 0.0)
    lg = jnp.dot(c, wc2_ref[...], preferred_element_type=jnp.float32) + bc2_ref[...]
    lmax = jnp.max(lg, axis=-1, keepdims=True)
    sh = lg - lmax
    logp_ref[...] = sh - jnp.log(jnp.sum(jnp.exp(sh), axis=-1, keepdims=True))
    ur_ref[...] = jnp.dot(h, wl1a_ref[...], preferred_element_type=jnp.float32)
    uc_ref[...] = jnp.dot(h, wl1b_ref[...], preferred_element_type=jnp.float32)


def _stage5(h, msg, W_c1, b_c1, bn_g, bn_b, W_c2, b_c2, W_l1a, W_l1b):
    return pl.pallas_call(
        _k_stage5,
        grid=(N // _BN,),
        in_specs=[_nspec2(TH), _nspec2(TH), _wspec((TH, 4 * HID)),
                  _wspec((1, 4 * HID)), _wspec((1, 4 * HID)), _wspec((1, 4 * HID)),
                  _wspec((4 * HID, OUT)), _wspec((1, OUT)),
                  _wspec((TH, HID)), _wspec((TH, HID))],
        out_specs=[_nspec2(TH), _nspec2(OUT), _nspec2(HID), _nspec2(HID)],
        out_shape=[jax.ShapeDtypeStruct((N, TH), jnp.float32),
                   jax.ShapeDtypeStruct((N, OUT), jnp.float32),
                   jax.ShapeDtypeStruct((N, HID), jnp.float32),
                   jax.ShapeDtypeStruct((N, HID), jnp.float32)],
    )(h, msg, W_c1, b_c1.reshape(1, 4 * HID), bn_g.reshape(1, 4 * HID),
      bn_b.reshape(1, 4 * HID), W_c2, b_c2.reshape(1, OUT), W_l1a, W_l1b)


def _k_link(ur_ref, uc_ref, bl1_ref, wl2_ref, bl2_ref, out_ref):
    v = jnp.maximum(ur_ref[...] + uc_ref[...] + bl1_ref[...], 0.0)
    out_ref[...] = jax.nn.sigmoid(
        jnp.sum(v * wl2_ref[...], axis=-1) + bl2_ref[0, 0, 0, 0])


def _link(ur_row, uc_col, b_l1, W_l2, b_l2):
    return pl.pallas_call(
        _k_link,
        grid=(_EG,),
        in_specs=[pl.BlockSpec((1, _EB, _EC, HID), lambda i: (i, 0, 0, 0))] * 2
        + [_wspec((1, 1, 1, HID)), _wspec((1, 1, 1, HID)), _wspec((1, 1, 1, 1))],
        out_specs=pl.BlockSpec((1, _EB, _EC), lambda i: (i, 0, 0)),
        out_shape=jax.ShapeDtypeStruct((_EG, _EB, _EC), jnp.float32),
    )(ur_row.reshape(_EG, _EB, _EC, HID), uc_col.reshape(_EG, _EB, _EC, HID),
      b_l1.reshape(1, 1, 1, HID), W_l2.reshape(1, 1, 1, HID),
      b_l2.reshape(1, 1, 1, 1))


_SC_CH = 128
_NSUB = 16
_FC = 128


_NP = 10240


def _sc_segsum_call(nchunks):
    mesh = plsc.VectorSubcoreMesh(core_axis_name="c", subcore_axis_name="s")
    rows_per_sub = _NP // _NSUB

    @functools.partial(
        pl.kernel, mesh=mesh,
        out_type=jax.ShapeDtypeStruct((4, _NP, _FC), jnp.float32),
        scratch_types=[
            pltpu.VMEM((nchunks, _SC_CH), jnp.int32),
            pltpu.VMEM((_SC_CH, _FC), jnp.float32),
            pltpu.VMEM_SHARED((_NP, _FC), jnp.float32),
        ],
    )
    def k(m_hbm, idx_hbm, zero_hbm, out_hbm, idx_v, rows_v, acc):
        c = lax.axis_index("c")
        s = lax.axis_index("s")
        pltpu.sync_copy(idx_hbm.at[s], idx_v)
        for j in range(2):
            fc = c * 2 + j
            pltpu.sync_copy(
                zero_hbm.at[pl.ds(s * rows_per_sub, rows_per_sub)],
                acc.at[pl.ds(s * rows_per_sub, rows_per_sub)])
            plsc.subcore_barrier()

            def body(t, carry):
                base = (s * nchunks + t) * _SC_CH
                pltpu.sync_copy(m_hbm.at[fc, pl.ds(base, _SC_CH)], rows_v)
                pltpu.sync_copy(rows_v, acc.at[idx_v.at[t]], add=True)
                return carry

            lax.fori_loop(0, nchunks, body, 0)
            plsc.subcore_barrier()
            pltpu.sync_copy(
                acc.at[pl.ds(s * rows_per_sub, rows_per_sub)],
                out_hbm.at[fc, pl.ds(s * rows_per_sub, rows_per_sub)])
            plsc.subcore_barrier()

    return k


def _sc_segsum(m, col):
    e = m.shape[0]
    ep = -(-e // (_NSUB * _SC_CH)) * (_NSUB * _SC_CH)
    nchunks = ep // (_NSUB * _SC_CH)
    pad = ep - e
    m4 = jnp.pad(m, ((0, pad), (0, 0))).reshape(ep, 4, _FC).transpose(1, 0, 2)
    idx3 = jnp.pad(col, (0, pad)).reshape(_NSUB, nchunks, _SC_CH)
    zero = jnp.zeros((_NP, _FC), jnp.float32)
    out4 = _sc_segsum_call(nchunks)(m4, idx3, zero)
    return out4[:, :N].transpose(1, 0, 2).reshape(N, TH)


def _ln(x, g, b):
    mu = jnp.mean(x, axis=-1, keepdims=True)
    var = jnp.var(x, axis=-1, keepdims=True)
    return (x - mu) / jnp.sqrt(var + 1e-5) * g + b


def _gat_edge(xp, a_s, a_d, row2, col2):
    xph = xp.reshape(N, HEADS, HID)
    asrc = jnp.sum(xph * a_s, axis=-1)
    adst = jnp.sum(xph * a_d, axis=-1)
    alpha = jax.nn.leaky_relu(asrc[row2] + adst[col2], 0.2)
    amax = jax.ops.segment_max(alpha, col2, num_segments=N)
    amax = jnp.where(jnp.isfinite(amax), amax, 0.0)
    e = jnp.exp(alpha - amax[col2])
    den = jax.ops.segment_sum(e, col2, num_segments=N)
    a = e / (den[col2] + 1e-16)
    msg = (a[:, :, None] * xph[row2]).reshape(row2.shape[0], TH)
    return jax.ops.segment_sum(msg, col2, num_segments=N)


def kernel(x, W_res, b_res, emb, W_g1, a_s1, a_d1, b_g1, ln1_g, ln1_b,
           W_g2, a_s2, a_d2, b_g2, ln2_g, ln2_b, W_p, b_p, W_bil, W_e, b_e,
           W_l1, b_l1, W_l2, b_l2, W_c1, b_c1, bn_g, bn_b, W_c2, b_c2,
           edge_index, y_label, target_mask):
    row = edge_index[0].astype(jnp.int32)
    col = edge_index[1].astype(jnp.int32)
    loop = jnp.arange(N, dtype=jnp.int32)
    row2 = jnp.concatenate([row, loop])
    col2 = jnp.concatenate([col, loop])

    xp1, x_proj = _stage1(x, W_res, b_res, W_g1)
    agg1 = _gat_edge(xp1, a_s1, a_d1, row2, col2)
    h1 = jax.nn.elu(_ln(agg1 + b_g1, ln1_g, ln1_b))

    xp2 = _stage2(h1, W_g2)
    agg2 = _gat_edge(xp2, a_s2, a_d2, row2, col2)
    h_base = jax.nn.elu(_ln(agg2 + b_g2, ln2_g, ln2_b) + x_proj)

    hp, q, xl1 = _stage3(h_base, W_p, b_p, W_bil, W_e, b_e)

    l0, l1 = _scores(q[row], hp[col])
    l0 = l0.reshape(E)
    l1 = l1.reshape(E)

    gkey = jax.random.key(42)
    final_w = jnp.zeros((E,), jnp.float32)
    active = target_mask.astype(jnp.float32)
    for hstep in range(HOPS):
        g = jax.random.gumbel(jax.random.fold_in(gkey, hstep), (E, 2))
        samp = ((l1 + g[:, 1]) > (l0 + g[:, 0])).astype(jnp.float32)
        cur = samp * active[row]
        final_w = jnp.maximum(final_w, cur)
        new_active = jax.ops.segment_sum(cur, col, num_segments=N)
        active = (new_active > 1e-5).astype(jnp.float32)
    weights = final_w

    wmax = jax.ops.segment_max(weights, col, num_segments=N)
    wmax = jnp.where(jnp.isfinite(wmax), wmax, 0.0)
    ew = jnp.exp(weights - wmax[col])
    dw = jax.ops.segment_sum(ew, col, num_segments=N)
    nw = ew / (dw[col] + 1e-16)

    msg1 = _sc_segsum(nw[:, None] * xl1[row], col)
    h2, xl2 = _stage4(h_base, msg1, W_e, b_e)
    msg2 = _sc_segsum(nw[:, None] * xl2[row], col)

    h_enh, logp, ur, uc = _stage5(
        h2, msg2, W_c1, b_c1, bn_g, bn_b, W_c2, b_c2,
        W_l1[:TH], W_l1[TH:])

    link = _link(ur[row], uc[col], b_l1, W_l2, b_l2).reshape(E)

    return (logp, link, h_enh, weights)
